# Initial kernel scaffold; baseline (speedup 1.0000x reference)
import jax
import jax.numpy as jnp
from jax.experimental import pallas as pl


def kernel(feats, coords, W, bn_gamma, bn_beta):
    raise NotImplementedError("write your pallas kernel here")


# speedup vs baseline: 3.2470x; 3.2470x over previous
import functools

import jax
import jax.numpy as jnp
from jax import lax
from jax.experimental import pallas as pl
from jax.experimental.pallas import tpu as pltpu
from jax.experimental.pallas import tpu_sc as plsc

GRID = 64
G2 = GRID + 2
K = 27
C = 128
NW = 32
TBL = G2 * G2 * G2
SEG = 8992
TBL_PAD = SEG * NW
CHUNK = 1664
NPAD = CHUNK * NW
SB = 128
SENTINEL = TBL - 1

OFFL = tuple(
    (dx * G2 + dy) * G2 + dz
    for dx in (-1, 0, 1) for dy in (-1, 0, 1) for dz in (-1, 0, 1)
)

_MESH = plsc.VectorSubcoreMesh(core_axis_name="c", subcore_axis_name="s")


def _worker_id():
    return lax.axis_index("s") * 2 + lax.axis_index("c")


def _gemm_body(f_ref, w_ref, o_ref):
    o_ref[0] = jnp.dot(f_ref[...], w_ref[0], preferred_element_type=jnp.float32)


def _gemm(feats_p, W):
    BM = 512
    nb = NPAD // BM
    return pl.pallas_call(
        _gemm_body,
        grid=(nb, K),
        in_specs=[
            pl.BlockSpec((BM, C), lambda i, k: (i, 0)),
            pl.BlockSpec((1, C, C), lambda i, k: (k, 0, 0)),
        ],
        out_specs=pl.BlockSpec((1, BM, C), lambda i, k: (k, i, 0)),
        out_shape=jax.ShapeDtypeStruct((K, NPAD, C), jnp.float32),
    )(feats_p, W)


def _table_body(keys_hbm, table_hbm, keys_v, tbl_v):
    wid = _worker_id()
    seg_base = wid * SEG
    fill = jnp.full((16,), NPAD - 1, jnp.int32)

    def fill_step(i, _):
        tbl_v[pl.ds(i * 16, 16)] = fill
        return 0

    lax.fori_loop(0, SEG // 16, fill_step, 0)
    pltpu.sync_copy(keys_hbm, keys_v)
    iota = lax.iota(jnp.int32, 16)

    def scat_step(i, _):
        kv = keys_v[pl.ds(i * 16, 16)]
        idx = kv - seg_base
        m = (idx >= 0) & (idx < SEG)
        idxc = jnp.where(m, idx, 0)
        plsc.store_scatter(tbl_v, [idxc], i * 16 + iota, mask=m)
        return 0

    lax.fori_loop(0, NPAD // 16, scat_step, 0)
    pltpu.sync_copy(tbl_v, table_hbm.at[pl.ds(seg_base, SEG)])


def _build_table(keys_p):
    f = pl.kernel(
        _table_body,
        out_type=jax.ShapeDtypeStruct((TBL_PAD,), jnp.int32),
        mesh=_MESH,
        compiler_params=pltpu.CompilerParams(needs_layout_passes=False),
        scratch_types=[
            pltpu.VMEM((NPAD,), jnp.int32),
            pltpu.VMEM((SEG,), jnp.int32),
        ],
    )
    return f(keys_p)


def _gacc_body(keys_hbm, table_hbm, p_hbm, out_hbm,
               keys_c, nk_v, gidx_v, acc, stag):
    wid = _worker_id()
    base = wid * CHUNK
    pltpu.sync_copy(keys_hbm.at[pl.ds(base, CHUNK)], keys_c)

    def sb_body(s, _):
        row0 = s * SB

        for k in range(K):
            def mk_idx(i, _):
                kv = keys_c[pl.ds(row0 + i * 16, 16)]
                nk = kv + OFFL[k]
                nk = jnp.minimum(jnp.maximum(nk, 0), TBL_PAD - 1)
                nk_v[pl.ds(i * 16, 16)] = nk
                return 0

            lax.fori_loop(0, SB // 16, mk_idx, 0, unroll=True)
            pltpu.sync_copy(table_hbm.at[nk_v], gidx_v)

            def mk_gidx(i, _):
                gidx_v[pl.ds(i * 16, 16)] = gidx_v[pl.ds(i * 16, 16)] + (k * NPAD)
                return 0

            lax.fori_loop(0, SB // 16, mk_gidx, 0, unroll=True)
            if k == 0:
                pltpu.sync_copy(p_hbm.at[gidx_v], acc)
            else:
                pltpu.sync_copy(p_hbm.at[gidx_v], stag)

                def add_row(r, _):
                    for c in range(C // 16):
                        plsc.addupdate(acc.at[r, pl.ds(c * 16, 16)],
                                       stag[r, pl.ds(c * 16, 16)])
                    return 0

                lax.fori_loop(0, SB, add_row, 0)

        pltpu.sync_copy(acc, out_hbm.at[pl.ds(base + row0, SB)])
        return 0

    lax.fori_loop(0, CHUNK // SB, sb_body, 0)


def _gather_acc(keys_p, table, p_flat):
    f = pl.kernel(
        _gacc_body,
        out_type=jax.ShapeDtypeStruct((NPAD, C), jnp.float32),
        mesh=_MESH,
        scratch_types=[
            pltpu.VMEM((CHUNK,), jnp.int32),
            pltpu.VMEM((SB,), jnp.int32),
            pltpu.VMEM((SB,), jnp.int32),
            pltpu.VMEM((SB, C), jnp.float32),
            pltpu.VMEM((SB, C), jnp.float32),
        ],
    )
    return f(keys_p, table, p_flat)


def _stats_body(x_ref, o_ref):
    i = pl.program_id(0)

    @pl.when(i == 0)
    def _():
        o_ref[...] = jnp.zeros_like(o_ref)

    x = x_ref[...]
    s = jnp.sum(x, axis=0)
    q = jnp.sum(x * x, axis=0)
    part = jnp.concatenate(
        [s[None], q[None], jnp.zeros((6, C), jnp.float32)], axis=0)
    o_ref[...] += part


def _bn_body(n, x_ref, st_ref, g_ref, b_ref, o_ref):
    s = st_ref[0]
    q = st_ref[1]
    mean = s / n
    var = q / n - mean * mean
    scale = g_ref[0] * lax.rsqrt(var + 1e-6)
    shift = b_ref[0] - mean * scale
    o_ref[...] = jnp.maximum(x_ref[...] * scale + shift, 0.0)


def _batchnorm(n, outp, gamma, beta):
    BR = 400
    nb = n // BR
    stats = pl.pallas_call(
        _stats_body,
        grid=(nb,),
        in_specs=[pl.BlockSpec((BR, C), lambda i: (i, 0))],
        out_specs=pl.BlockSpec((8, C), lambda i: (0, 0)),
        out_shape=jax.ShapeDtypeStruct((8, C), jnp.float32),
    )(outp)
    return pl.pallas_call(
        functools.partial(_bn_body, float(n)),
        grid=(nb,),
        in_specs=[
            pl.BlockSpec((BR, C), lambda i: (i, 0)),
            pl.BlockSpec((8, C), lambda i: (0, 0)),
            pl.BlockSpec((1, C), lambda i: (0, 0)),
            pl.BlockSpec((1, C), lambda i: (0, 0)),
        ],
        out_specs=pl.BlockSpec((BR, C), lambda i: (i, 0)),
        out_shape=jax.ShapeDtypeStruct((n, C), jnp.float32),
    )(outp, stats, gamma.reshape(1, C), beta.reshape(1, C))


def kernel(feats, coords, W, bn_gamma, bn_beta):
    n = feats.shape[0]
    feats_p = jnp.zeros((NPAD, C), jnp.float32).at[:n].set(feats)
    key = ((coords[:, 0] + 1) * G2 + (coords[:, 1] + 1)) * G2 + (coords[:, 2] + 1)
    keys_p = jnp.full((NPAD,), SENTINEL, jnp.int32).at[:n].set(key.astype(jnp.int32))

    p = _gemm(feats_p, W)
    table = _build_table(keys_p)
    outp = _gather_acc(keys_p, table, p.reshape(K * NPAD, C))
    return _batchnorm(n, outp, bn_gamma, bn_beta)
